# baseline (speedup 1.0000x reference)
import functools

import jax
import jax.numpy as jnp
from jax import lax
from jax.experimental import pallas as pl
from jax.experimental.pallas import tpu as pltpu
from jax.experimental.pallas import tpu_sc as plsc

_DN = (((1,), (0,)), ((), ()))


def _enc_kernel(x_ref, wi_ref, bi_ref, we_ref, be_ref, o_ref):
    h = lax.dot_general(x_ref[...], wi_ref[...], _DN,
                        preferred_element_type=jnp.float32) + bi_ref[...]
    o_ref[...] = lax.dot_general(h, we_ref[...], _DN,
                                 preferred_element_type=jnp.float32) + be_ref[...]


def _encode(roi, W_in, b_in, W_enc, b_enc, bn=512):
    n, k = roi.shape
    m = W_in.shape[1]
    return pl.pallas_call(
        _enc_kernel,
        grid=(n // bn,),
        in_specs=[pl.BlockSpec((bn, k), lambda i: (i, 0)),
                  pl.BlockSpec((k, m), lambda i: (0, 0)),
                  pl.BlockSpec((1, m), lambda i: (0, 0)),
                  pl.BlockSpec((m, m), lambda i: (0, 0)),
                  pl.BlockSpec((1, m), lambda i: (0, 0))],
        out_specs=pl.BlockSpec((bn, m), lambda i: (i, 0)),
        out_shape=jax.ShapeDtypeStruct((n, m), jnp.float32),
    )(roi, W_in, b_in.reshape(1, m), W_enc, b_enc.reshape(1, m))


def _sc_gather(table, idx):
    info = plsc.get_sparse_core_info()
    nw = info.num_cores * info.num_subcores
    b = idx.shape[0]
    d = table.shape[1]
    b_per_w = b // nw
    chunk = 32
    nch = b_per_w // chunk
    mesh = plsc.VectorSubcoreMesh(core_axis_name="c", subcore_axis_name="s")

    @functools.partial(
        pl.kernel, mesh=mesh,
        out_type=jax.ShapeDtypeStruct((b, d), jnp.float32),
        scratch_types=[pltpu.VMEM((chunk,), jnp.int32),
                       pltpu.VMEM((chunk,), jnp.int32),
                       pltpu.VMEM((chunk, d), jnp.float32),
                       pltpu.VMEM((chunk, d), jnp.float32),
                       pltpu.SemaphoreType.DMA,
                       pltpu.SemaphoreType.DMA],
    )
    def gather_k(table_hbm, idx_hbm, out_hbm, idx0, idx1, rows0, rows1,
                 sem0, sem1):
        wid = lax.axis_index("s") * info.num_cores + lax.axis_index("c")
        base = wid * b_per_w
        idx_bufs = (idx0, idx1)
        row_bufs = (rows0, rows1)
        sems = (sem0, sem1)
        pltpu.sync_copy(idx_hbm.at[pl.ds(base, chunk)], idx0)
        cps = [pltpu.async_copy(table_hbm.at[idx0], rows0, sem0), None]
        for c in range(nch):
            nxt = (c + 1) % 2
            if c + 1 < nch:
                pltpu.sync_copy(
                    idx_hbm.at[pl.ds(base + (c + 1) * chunk, chunk)],
                    idx_bufs[nxt])
                cps[nxt] = pltpu.async_copy(
                    table_hbm.at[idx_bufs[nxt]], row_bufs[nxt], sems[nxt])
            cps[c % 2].wait()
            pltpu.sync_copy(row_bufs[c % 2],
                            out_hbm.at[pl.ds(base + c * chunk, chunk)])

    return gather_k(table, idx)


def _dec_kernel(zq_ref, ze_ref, roi_ref, wd_ref, bd_ref, wo_ref, bo_ref,
                recon_ref, part_ref):
    zq = zq_ref[...]
    decoded = lax.dot_general(zq, wd_ref[...], _DN,
                              preferred_element_type=jnp.float32) + bd_ref[...]
    recon = lax.dot_general(decoded, wo_ref[...], _DN,
                            preferred_element_type=jnp.float32) + bo_ref[...]
    recon_ref[...] = recon
    sq = jnp.sum((recon - roi_ref[...]) ** 2)
    emb_d = zq - ze_ref[...]
    emb = jnp.sum(emb_d * emb_d)
    lanes = lax.broadcasted_iota(jnp.int32, (1, 1, 128), 2)
    part_ref[...] = (jnp.where(lanes == 0, sq, 0.0)
                     + jnp.where(lanes == 1, emb, 0.0))


def _decode(z_q, z_e, roi, W_dec, b_dec, W_out, b_out, bn=512):
    n, m = z_q.shape
    k = W_out.shape[1]
    nb = n // bn
    recon, parts = pl.pallas_call(
        _dec_kernel,
        grid=(nb,),
        in_specs=[pl.BlockSpec((bn, m), lambda i: (i, 0)),
                  pl.BlockSpec((bn, m), lambda i: (i, 0)),
                  pl.BlockSpec((bn, k), lambda i: (i, 0)),
                  pl.BlockSpec((m, m), lambda i: (0, 0)),
                  pl.BlockSpec((1, m), lambda i: (0, 0)),
                  pl.BlockSpec((m, k), lambda i: (0, 0)),
                  pl.BlockSpec((1, k), lambda i: (0, 0))],
        out_specs=[pl.BlockSpec((bn, k), lambda i: (i, 0)),
                   pl.BlockSpec((1, 1, 128), lambda i: (i, 0, 0))],
        out_shape=[jax.ShapeDtypeStruct((n, k), jnp.float32),
                   jax.ShapeDtypeStruct((nb, 1, 128), jnp.float32)],
        compiler_params=pltpu.CompilerParams(vmem_limit_bytes=100 * 2 ** 20),
    )(z_q, z_e, roi, W_dec, b_dec.reshape(1, m), W_out, b_out.reshape(1, k))
    return recon, parts


def kernel(roi_feats, W_in, b_in, W_enc, b_enc, codebook, W_dec, b_dec, W_out, b_out):
    commitment_cost = 0.25
    z_e = _encode(roi_feats, W_in, b_in, W_enc, b_enc)
    z_e_flat = z_e.reshape(-1, z_e.shape[-1])
    d = (jnp.sum(z_e_flat ** 2, axis=1, keepdims=True)
         - 2.0 * (z_e_flat @ codebook.T)
         + jnp.sum(codebook ** 2, axis=1))
    min_encoding_indices = jnp.argmin(d, axis=-1)
    n, k = roi_feats.shape
    m = z_e.shape[1]
    half = n // 2
    zq0 = _sc_gather(codebook, min_encoding_indices[:half])
    zq1 = _sc_gather(codebook, min_encoding_indices[half:])
    rec0, p0 = _decode(zq0, z_e[:half], roi_feats[:half], W_dec, b_dec,
                       W_out, b_out)
    rec1, p1 = _decode(zq1, z_e[half:], roi_feats[half:], W_dec, b_dec,
                       W_out, b_out)
    z_q = jnp.concatenate([zq0, zq1], axis=0)
    recon = jnp.concatenate([rec0, rec1], axis=0)
    parts = p0 + p1
    recon_loss = jnp.sum(parts[:, 0, 0]) / (n * k)
    embedding_loss = jnp.sum(parts[:, 0, 1]) / (n * m)
    commitment_loss = embedding_loss
    vq_loss = recon_loss + embedding_loss + commitment_cost * commitment_loss
    return (min_encoding_indices, z_q, recon, vq_loss, recon_loss,
            embedding_loss, commitment_loss)


# speedup vs baseline: 1.3085x; 1.3085x over previous
import functools

import jax
import jax.numpy as jnp
from jax import lax
from jax.experimental import pallas as pl
from jax.experimental.pallas import tpu as pltpu
from jax.experimental.pallas import tpu_sc as plsc

_DN = (((1,), (0,)), ((), ()))


def _enc_kernel(x_ref, wi_ref, bi_ref, we_ref, be_ref, o_ref):
    h = lax.dot_general(x_ref[...], wi_ref[...], _DN,
                        preferred_element_type=jnp.float32) + bi_ref[...]
    o_ref[...] = lax.dot_general(h, we_ref[...], _DN,
                                 preferred_element_type=jnp.float32) + be_ref[...]


def _encode(roi, W_in, b_in, W_enc, b_enc, bn=512):
    n, k = roi.shape
    m = W_in.shape[1]
    return pl.pallas_call(
        _enc_kernel,
        grid=(n // bn,),
        in_specs=[pl.BlockSpec((bn, k), lambda i: (i, 0)),
                  pl.BlockSpec((k, m), lambda i: (0, 0)),
                  pl.BlockSpec((1, m), lambda i: (0, 0)),
                  pl.BlockSpec((m, m), lambda i: (0, 0)),
                  pl.BlockSpec((1, m), lambda i: (0, 0))],
        out_specs=pl.BlockSpec((bn, m), lambda i: (i, 0)),
        out_shape=jax.ShapeDtypeStruct((n, m), jnp.float32),
        compiler_params=pltpu.CompilerParams(vmem_limit_bytes=63 * 2 ** 20),
    )(roi, W_in, b_in.reshape(1, m), W_enc, b_enc.reshape(1, m))


def _sc_gather(table, idx):
    info = plsc.get_sparse_core_info()
    nw = info.num_cores * info.num_subcores
    b = idx.shape[0]
    d = table.shape[1]
    b_per_w = b // nw
    chunk = 32
    nch = b_per_w // chunk
    mesh = plsc.VectorSubcoreMesh(core_axis_name="c", subcore_axis_name="s")

    @functools.partial(
        pl.kernel, mesh=mesh,
        out_type=jax.ShapeDtypeStruct((b, d), jnp.float32),
        scratch_types=[pltpu.VMEM((chunk,), jnp.int32),
                       pltpu.VMEM((chunk,), jnp.int32),
                       pltpu.VMEM((chunk, d), jnp.float32),
                       pltpu.VMEM((chunk, d), jnp.float32),
                       pltpu.SemaphoreType.DMA,
                       pltpu.SemaphoreType.DMA],
    )
    def gather_k(table_hbm, idx_hbm, out_hbm, idx0, idx1, rows0, rows1,
                 sem0, sem1):
        wid = lax.axis_index("s") * info.num_cores + lax.axis_index("c")
        base = wid * b_per_w
        idx_bufs = (idx0, idx1)
        row_bufs = (rows0, rows1)
        sems = (sem0, sem1)
        pltpu.sync_copy(idx_hbm.at[pl.ds(base, chunk)], idx0)
        cps = [pltpu.async_copy(table_hbm.at[idx0], rows0, sem0), None]
        for c in range(nch):
            nxt = (c + 1) % 2
            if c + 1 < nch:
                pltpu.sync_copy(
                    idx_hbm.at[pl.ds(base + (c + 1) * chunk, chunk)],
                    idx_bufs[nxt])
                cps[nxt] = pltpu.async_copy(
                    table_hbm.at[idx_bufs[nxt]], row_bufs[nxt], sems[nxt])
            cps[c % 2].wait()
            pltpu.sync_copy(row_bufs[c % 2],
                            out_hbm.at[pl.ds(base + c * chunk, chunk)])

    return gather_k(table, idx)


def _dec_kernel(nb, nk, nm, zq_ref, ze_ref, roi_ref, wd_ref, bd_ref, wo_ref,
                bo_ref, recon_ref, loss_ref, acc_ref):
    i = pl.program_id(0)
    zq = zq_ref[...]
    decoded = lax.dot_general(zq, wd_ref[...], _DN,
                              preferred_element_type=jnp.float32) + bd_ref[...]
    recon = lax.dot_general(decoded, wo_ref[...], _DN,
                            preferred_element_type=jnp.float32) + bo_ref[...]
    recon_ref[...] = recon
    sq = jnp.sum((recon - roi_ref[...]) ** 2)
    emb_d = zq - ze_ref[...]
    emb = jnp.sum(emb_d * emb_d)
    lanes = lax.broadcasted_iota(jnp.int32, (1, 128), 1)
    part = jnp.where(lanes == 0, sq, 0.0) + jnp.where(lanes == 1, emb, 0.0)

    @pl.when(i == 0)
    def _():
        acc_ref[...] = jnp.zeros_like(acc_ref)

    acc_ref[...] += part

    @pl.when(i == nb - 1)
    def _():
        acc = acc_ref[...]
        recon_loss = acc[0, 0] / nk
        embedding_loss = acc[0, 1] / nm
        vq_loss = recon_loss + embedding_loss + 0.25 * embedding_loss
        loss_ref[...] = (jnp.where(lanes == 0, vq_loss, 0.0)
                         + jnp.where(lanes == 1, recon_loss, 0.0)
                         + jnp.where(lanes == 2, embedding_loss, 0.0))


def _decode(z_q, z_e, roi, W_dec, b_dec, W_out, b_out, bn=512):
    n, m = z_q.shape
    k = W_out.shape[1]
    nb = n // bn
    recon, losses = pl.pallas_call(
        functools.partial(_dec_kernel, nb, float(n * k), float(n * m)),
        grid=(nb,),
        in_specs=[pl.BlockSpec((bn, m), lambda i: (i, 0)),
                  pl.BlockSpec((bn, m), lambda i: (i, 0)),
                  pl.BlockSpec((bn, k), lambda i: (i, 0)),
                  pl.BlockSpec((m, m), lambda i: (0, 0)),
                  pl.BlockSpec((1, m), lambda i: (0, 0)),
                  pl.BlockSpec((m, k), lambda i: (0, 0)),
                  pl.BlockSpec((1, k), lambda i: (0, 0))],
        out_specs=[pl.BlockSpec((bn, k), lambda i: (i, 0)),
                   pl.BlockSpec((1, 128), lambda i: (0, 0))],
        out_shape=[jax.ShapeDtypeStruct((n, k), jnp.float32),
                   jax.ShapeDtypeStruct((1, 128), jnp.float32)],
        scratch_shapes=[pltpu.VMEM((1, 128), jnp.float32)],
        compiler_params=pltpu.CompilerParams(vmem_limit_bytes=63 * 2 ** 20),
    )(z_q, z_e, roi, W_dec, b_dec.reshape(1, m), W_out, b_out.reshape(1, k))
    return recon, losses


def kernel(roi_feats, W_in, b_in, W_enc, b_enc, codebook, W_dec, b_dec, W_out, b_out):
    commitment_cost = 0.25
    z_e = _encode(roi_feats, W_in, b_in, W_enc, b_enc)
    z_e_flat = z_e.reshape(-1, z_e.shape[-1])
    d = (jnp.sum(z_e_flat ** 2, axis=1, keepdims=True)
         - 2.0 * (z_e_flat @ codebook.T)
         + jnp.sum(codebook ** 2, axis=1))
    min_encoding_indices = jnp.argmin(d, axis=-1)
    del commitment_cost
    z_q = _sc_gather(codebook, min_encoding_indices)
    recon, losses = _decode(z_q, z_e, roi_feats, W_dec, b_dec, W_out, b_out)
    vq_loss = losses[0, 0]
    recon_loss = losses[0, 1]
    embedding_loss = losses[0, 2]
    commitment_loss = embedding_loss
    return (min_encoding_indices, z_q, recon, vq_loss, recon_loss,
            embedding_loss, commitment_loss)


# speedup vs baseline: 1.3094x; 1.0007x over previous
import functools

import jax
import jax.numpy as jnp
from jax import lax
from jax.experimental import pallas as pl
from jax.experimental.pallas import tpu as pltpu
from jax.experimental.pallas import tpu_sc as plsc

_DN = (((1,), (0,)), ((), ()))


def _enc_kernel(x_ref, wi_ref, bi_ref, we_ref, be_ref, o_ref):
    h = lax.dot_general(x_ref[...], wi_ref[...], _DN,
                        preferred_element_type=jnp.float32) + bi_ref[...]
    o_ref[...] = lax.dot_general(h, we_ref[...], _DN,
                                 preferred_element_type=jnp.float32) + be_ref[...]


def _encode(roi, W_in, b_in, W_enc, b_enc, bn=512):
    n, k = roi.shape
    m = W_in.shape[1]
    return pl.pallas_call(
        _enc_kernel,
        grid=(n // bn,),
        in_specs=[pl.BlockSpec((bn, k), lambda i: (i, 0)),
                  pl.BlockSpec((k, m), lambda i: (0, 0)),
                  pl.BlockSpec((1, m), lambda i: (0, 0)),
                  pl.BlockSpec((m, m), lambda i: (0, 0)),
                  pl.BlockSpec((1, m), lambda i: (0, 0))],
        out_specs=pl.BlockSpec((bn, m), lambda i: (i, 0)),
        out_shape=jax.ShapeDtypeStruct((n, m), jnp.float32),
        compiler_params=pltpu.CompilerParams(vmem_limit_bytes=63 * 2 ** 20),
    )(roi, W_in, b_in.reshape(1, m), W_enc, b_enc.reshape(1, m))


def _sc_gather(table, idx):
    info = plsc.get_sparse_core_info()
    nw = info.num_cores * info.num_subcores
    b = idx.shape[0]
    d = table.shape[1]
    b_per_w = b // nw
    chunk = 32
    nch = b_per_w // chunk
    mesh = plsc.VectorSubcoreMesh(core_axis_name="c", subcore_axis_name="s")

    @functools.partial(
        pl.kernel, mesh=mesh,
        out_type=jax.ShapeDtypeStruct((b, d), jnp.float32),
        scratch_types=[pltpu.VMEM((chunk,), jnp.int32),
                       pltpu.VMEM((chunk,), jnp.int32),
                       pltpu.VMEM((chunk, d), jnp.float32),
                       pltpu.VMEM((chunk, d), jnp.float32),
                       pltpu.SemaphoreType.DMA,
                       pltpu.SemaphoreType.DMA],
    )
    def gather_k(table_hbm, idx_hbm, out_hbm, idx0, idx1, rows0, rows1,
                 sem0, sem1):
        wid = lax.axis_index("s") * info.num_cores + lax.axis_index("c")
        base = wid * b_per_w
        idx_bufs = (idx0, idx1)
        row_bufs = (rows0, rows1)
        sems = (sem0, sem1)
        pltpu.sync_copy(idx_hbm.at[pl.ds(base, chunk)], idx0)
        cps = [pltpu.async_copy(table_hbm.at[idx0], rows0, sem0), None]
        for c in range(nch):
            nxt = (c + 1) % 2
            if c + 1 < nch:
                pltpu.sync_copy(
                    idx_hbm.at[pl.ds(base + (c + 1) * chunk, chunk)],
                    idx_bufs[nxt])
                cps[nxt] = pltpu.async_copy(
                    table_hbm.at[idx_bufs[nxt]], row_bufs[nxt], sems[nxt])
            cps[c % 2].wait()
            pltpu.sync_copy(row_bufs[c % 2],
                            out_hbm.at[pl.ds(base + c * chunk, chunk)])

    return gather_k(table, idx)


def _dec_kernel(nb, nk, nm, zq_ref, ze_ref, roi_ref, wd_ref, bd_ref, wo_ref,
                bo_ref, recon_ref, loss_ref, acc_ref):
    i = pl.program_id(0)
    zq = zq_ref[...]
    decoded = lax.dot_general(zq, wd_ref[...], _DN,
                              preferred_element_type=jnp.float32) + bd_ref[...]
    recon = lax.dot_general(decoded, wo_ref[...], _DN,
                            preferred_element_type=jnp.float32) + bo_ref[...]
    recon_ref[...] = recon
    sq = jnp.sum((recon - roi_ref[...]) ** 2)
    emb_d = zq - ze_ref[...]
    emb = jnp.sum(emb_d * emb_d)
    lanes = lax.broadcasted_iota(jnp.int32, (1, 128), 1)
    part = jnp.where(lanes == 0, sq, 0.0) + jnp.where(lanes == 1, emb, 0.0)

    @pl.when(i == 0)
    def _():
        acc_ref[...] = jnp.zeros_like(acc_ref)

    acc_ref[...] += part

    @pl.when(i == nb - 1)
    def _():
        acc = acc_ref[...]
        recon_loss = acc[0, 0] / nk
        embedding_loss = acc[0, 1] / nm
        vq_loss = recon_loss + embedding_loss + 0.25 * embedding_loss
        loss_ref[...] = (jnp.where(lanes == 0, vq_loss, 0.0)
                         + jnp.where(lanes == 1, recon_loss, 0.0)
                         + jnp.where(lanes == 2, embedding_loss, 0.0))


def _decode(z_q, z_e, roi, W_dec, b_dec, W_out, b_out, bn=512):
    n, m = z_q.shape
    k = W_out.shape[1]
    nb = n // bn
    recon, losses = pl.pallas_call(
        functools.partial(_dec_kernel, nb, float(n * k), float(n * m)),
        grid=(nb,),
        in_specs=[pl.BlockSpec((bn, m), lambda i: (i, 0)),
                  pl.BlockSpec((bn, m), lambda i: (i, 0)),
                  pl.BlockSpec((bn, k), lambda i: (i, 0)),
                  pl.BlockSpec((m, m), lambda i: (0, 0)),
                  pl.BlockSpec((1, m), lambda i: (0, 0)),
                  pl.BlockSpec((m, k), lambda i: (0, 0)),
                  pl.BlockSpec((1, k), lambda i: (0, 0))],
        out_specs=[pl.BlockSpec((bn, k), lambda i: (i, 0)),
                   pl.BlockSpec((1, 128), lambda i: (0, 0))],
        out_shape=[jax.ShapeDtypeStruct((n, k), jnp.float32),
                   jax.ShapeDtypeStruct((1, 128), jnp.float32)],
        scratch_shapes=[pltpu.VMEM((1, 128), jnp.float32)],
        compiler_params=pltpu.CompilerParams(vmem_limit_bytes=63 * 2 ** 20),
    )(z_q, z_e, roi, W_dec, b_dec.reshape(1, m), W_out, b_out.reshape(1, k))
    return recon, losses


def kernel(roi_feats, W_in, b_in, W_enc, b_enc, codebook, W_dec, b_dec, W_out, b_out):
    z_e = _encode(roi_feats, W_in, b_in, W_enc, b_enc)
    z_e_flat = z_e.reshape(-1, z_e.shape[-1])
    d = (jnp.sum(z_e_flat ** 2, axis=1, keepdims=True)
         - 2.0 * (z_e_flat @ codebook.T)
         + jnp.sum(codebook ** 2, axis=1))
    min_encoding_indices = jnp.argmin(d, axis=-1)
    z_q = _sc_gather(codebook, min_encoding_indices)
    recon, losses = _decode(z_q, z_e, roi_feats, W_dec, b_dec, W_out, b_out)
    vq_loss = losses[0, 0]
    recon_loss = losses[0, 1]
    embedding_loss = losses[0, 2]
    commitment_loss = embedding_loss
    return (min_encoding_indices, z_q, recon, vq_loss, recon_loss,
            embedding_loss, commitment_loss)
